# baseline (speedup 1.0000x reference)
import jax
import jax.numpy as jnp
from jax import lax
from jax.experimental import pallas as pl
from jax.experimental.pallas import tpu as pltpu
from jax.experimental.pallas import tpu_sc as plsc

_B = 4096 * 200
_D = 64
_NC = 2
_NS = 16
_NW = _NC * _NS
_BPW = _B // _NW
_C = 512
_NCH = _BPW // _C


def _gather_kernel(x_hbm, table_hbm, out_hbm, idx_v, rows_v, sem):
    wid = lax.axis_index("s") * _NC + lax.axis_index("c")
    base = wid * _BPW

    def body(g, carry):
        off = base + g * _C
        pltpu.sync_copy(x_hbm.at[pl.ds(off, _C)], idx_v)
        pltpu.async_copy(table_hbm.at[idx_v], rows_v, sem).wait()
        pltpu.sync_copy(rows_v, out_hbm.at[pl.ds(off, _C)])
        return carry

    lax.fori_loop(0, _NCH, body, 0)


@jax.jit
def kernel(x, table):
    x_flat = x.reshape(-1).astype(jnp.int32)
    mesh = plsc.VectorSubcoreMesh(core_axis_name="c", subcore_axis_name="s")
    out = pl.kernel(
        _gather_kernel,
        out_type=jax.ShapeDtypeStruct((_B, _D), jnp.float32),
        mesh=mesh,
        scratch_types=[
            pltpu.VMEM((_C,), jnp.int32),
            pltpu.VMEM((_C, _D), jnp.float32),
            pltpu.SemaphoreType.DMA,
        ],
        compiler_params=pltpu.CompilerParams(use_tc_tiling_on_sc=False),
    )(x_flat, table)
    return out.reshape(x.shape[0], x.shape[1], _D)


# speedup vs baseline: 1.0424x; 1.0424x over previous
import jax
import jax.numpy as jnp
from jax import lax
from jax.experimental import pallas as pl
from jax.experimental.pallas import tpu as pltpu
from jax.experimental.pallas import tpu_sc as plsc

_B = 4096 * 200
_D = 64
_NC = 2
_NS = 16
_NW = _NC * _NS
_BPW = _B // _NW
_C = 256
_NCH = _BPW // _C
_NBUF = 4


def _gather_kernel(x_hbm, table_hbm, out_hbm, idx_v, rows, sems_g, sems_w):
    wid = lax.axis_index("s") * _NC + lax.axis_index("c")
    base = wid * _BPW

    pltpu.sync_copy(x_hbm.at[pl.ds(wid * _NCH, _NCH)], idx_v)

    def issue_gather(g, b):
        pltpu.async_copy(table_hbm.at[idx_v.at[g]], rows[b], sems_g[b])

    def wait_gather(g, b):
        pltpu.make_async_copy(table_hbm.at[idx_v.at[g]], rows[b], sems_g[b]).wait()

    def issue_wb(g, b):
        pltpu.async_copy(rows[b], out_hbm.at[pl.ds(base + g * _C, _C)], sems_w[b])

    def wait_wb(g, b):
        pltpu.make_async_copy(
            rows[b], out_hbm.at[pl.ds(base + g * _C, _C)], sems_w[b]).wait()

    for b in range(_NBUF):
        issue_gather(b, b)

    wait_gather(0, 0)
    issue_wb(0, 0)

    def body(t, carry):
        for j in range(_NBUF):
            g = 1 + t * _NBUF + j
            bp = j
            b = (1 + j) % _NBUF
            wait_wb(g - 1, bp)
            issue_gather(g - 1 + _NBUF, bp)
            wait_gather(g, b)
            issue_wb(g, b)
        return carry

    n_main = (_NCH - _NBUF) // _NBUF
    lax.fori_loop(0, n_main, body, 0)

    for g in range(_NCH - _NBUF + 1, _NCH):
        wait_wb(g - 1, (g - 1) % _NBUF)
        wait_gather(g, g % _NBUF)
        issue_wb(g, g % _NBUF)

    wait_wb(_NCH - 1, (_NCH - 1) % _NBUF)


@jax.jit
def kernel(x, table):
    x_2d = x.reshape(_NW * _NCH, _C).astype(jnp.int32)
    mesh = plsc.VectorSubcoreMesh(core_axis_name="c", subcore_axis_name="s")
    out = pl.kernel(
        _gather_kernel,
        out_type=jax.ShapeDtypeStruct((_B, _D), jnp.float32),
        mesh=mesh,
        scratch_types=[
            pltpu.VMEM((_NCH, _C), jnp.int32),
            [pltpu.VMEM((_C, _D), jnp.float32) for _ in range(_NBUF)],
            [pltpu.SemaphoreType.DMA for _ in range(_NBUF)],
            [pltpu.SemaphoreType.DMA for _ in range(_NBUF)],
        ],
        compiler_params=pltpu.CompilerParams(use_tc_tiling_on_sc=False),
    )(x_2d, table)
    return out.reshape(x.shape[0], x.shape[1], _D)
